# baseline (speedup 1.0000x reference)
import functools

import jax
import jax.numpy as jnp
from jax import lax
from jax.experimental import pallas as pl
from jax.experimental.pallas import tpu as pltpu
from jax.experimental.pallas import tpu_sc as plsc

B = 4096
L = 200
D = 128

_info = plsc.get_sparse_core_info()
_NC = _info.num_cores
_NS = _info.num_subcores
_NW = _NC * _NS
_BPW = B // _NW

_mesh = plsc.VectorSubcoreMesh(core_axis_name="c", subcore_axis_name="s")


@functools.partial(
    pl.kernel,
    mesh=_mesh,
    out_type=jax.ShapeDtypeStruct((B, D * L), jnp.float32),
    scratch_types=[
        pltpu.VMEM((L,), jnp.int32),
        pltpu.VMEM((L,), jnp.int32),
        pltpu.VMEM((L, D), jnp.float32),
        pltpu.VMEM((L, D), jnp.float32),
        pltpu.VMEM((D * L,), jnp.float32),
        pltpu.VMEM((D * L,), jnp.float32),
        pltpu.SemaphoreType.DMA,
        pltpu.SemaphoreType.DMA,
        pltpu.SemaphoreType.DMA,
        pltpu.SemaphoreType.DMA,
    ],
    compiler_params=pltpu.CompilerParams(needs_layout_passes=False),
)
def _sc_fused(x_hbm, table_hbm, fmap_hbm,
              idx0, idx1, emb0, emb1, out0, out1, gs0, gs1, os0, os1):
    wid = lax.axis_index("s") * _NC + lax.axis_index("c")
    base = wid * _BPW
    slots = ((idx0, emb0, out0, gs0, os0), (idx1, emb1, out1, gs1, os1))
    dbase = [(lax.iota(jnp.int32, 16) + d0) * L for d0 in range(0, D, 16)]

    def issue(j, s):
        idx_v, emb_v, _, gsem, _ = slots[s]
        pltpu.sync_copy(x_hbm.at[base + j], idx_v)
        pltpu.async_copy(table_hbm.at[idx_v], emb_v, gsem)

    def transpose(emb_v, out_v):
        def lstep(l, carry):
            for k in range(D // 16):
                vals = emb_v[l, pl.ds(k * 16, 16)]
                plsc.store_scatter(out_v, [dbase[k] + l], vals)
            return carry
        lax.fori_loop(0, L, lstep, 0)

    issue(0, 0)
    issue(1, 1)

    def pair(i, carry):
        for s in (0, 1):
            j = 2 * i + s
            b = base + j
            idx_v, emb_v, out_v, gsem, osem = slots[s]
            pltpu.make_async_copy(table_hbm.at[idx_v], emb_v, gsem).wait()

            @pl.when(j >= 2)
            def _():
                pltpu.make_async_copy(out_v, fmap_hbm.at[b], osem).wait()

            transpose(emb_v, out_v)
            pltpu.async_copy(out_v, fmap_hbm.at[b], osem)

            @pl.when(j + 2 < _BPW)
            def _():
                issue(j + 2, s)
        return carry

    lax.fori_loop(0, _BPW // 2, pair, 0)

    for s in (0, 1):
        _, _, out_v, _, osem = slots[s]
        pltpu.make_async_copy(out_v, fmap_hbm.at[base], osem).wait()


def _len_body(x_ref, len_ref):
    xr = x_ref[...]
    len_ref[...] = jnp.sum((xr != 0).astype(jnp.int32), axis=2, keepdims=True)


_lengths = pl.pallas_call(
    _len_body,
    grid=(1,),
    in_specs=[pl.BlockSpec((B, 1, L), lambda i: (0, 0, 0))],
    out_specs=pl.BlockSpec((B, 1, 1), lambda i: (0, 0, 0)),
    out_shape=jax.ShapeDtypeStruct((B, 1, 1), jnp.int32),
)


def kernel(x, table):
    x = x.astype(jnp.int32)
    fmap = _sc_fused(x, table)
    lens = _lengths(x.reshape(B, 1, L))
    return fmap.reshape(B, D, L), lens.reshape(B)
